# baseline (speedup 1.0000x reference)
import functools

import jax
import jax.numpy as jnp
from jax import lax
from jax.experimental import pallas as pl
from jax.experimental.pallas import tpu as pltpu
from jax.experimental.pallas import tpu_sc as plsc

_B = 1024
_S = 200
_DIM = 128
_HALF = _DIM // 2
_BASE = 10000

_NC = 2
_NS = 16
_NW = _NC * _NS
_BPW = _B // _NW
_G0 = 128
_G1 = _S - _G0
_TPW = _BPW


def _sincos():
    inv_freq = 1.0 / (_BASE ** (jnp.arange(0, _HALF, dtype=jnp.float32) / _HALF))
    angles = jnp.arange(_S, dtype=jnp.float32)[:, None] * inv_freq[None, :]
    return jnp.sin(angles), jnp.cos(angles)


def _body(ids0_ref, ids1_ref, table_ref, sin_ref, cos_ref, out_ref,
          idx0_v, idx1_v, rows0_v, rows1_v, sin_v, cos_v,
          gsem0, gsem1, wsem0, wsem1):
    wid = lax.axis_index("s") * _NC + lax.axis_index("c")
    base = wid * _BPW

    pltpu.sync_copy(sin_ref, sin_v)
    pltpu.sync_copy(cos_ref, cos_v)
    pltpu.sync_copy(ids0_ref.at[pl.ds(base, _BPW)], idx0_v)
    pltpu.sync_copy(ids1_ref.at[pl.ds(base, _BPW)], idx1_v)

    cfg = (
        (idx0_v, rows0_v, _G0, 0, gsem0, wsem0),
        (idx1_v, rows1_v, _G1, _G0, gsem1, wsem1),
    )

    def local_batches(t):
        q = t >> 1
        return 2 * q, 2 * q + 1

    def start_gather(t):
        idx_v, rows_v, g, _, gsem, _ = cfg[t & 1]
        k0, k1 = local_batches(t)
        return [
            pltpu.async_copy(table_ref.at[idx_v.at[k]],
                             rows_v.at[pl.ds(c * g, g)], gsem)
            for c, k in enumerate((k0, k1))
        ]

    def compute(t):
        _, rows_v, g, pos0, _, _ = cfg[t & 1]

        @plsc.parallel_loop(0, g, step=1, unroll=2)
        def row_body(i):
            for j in range(_HALF // 16):
                lo = pl.ds(j * 16, 16)
                hi = pl.ds(_HALF + j * 16, 16)
                cosv = cos_v[pos0 + i, lo]
                sinv = sin_v[pos0 + i, lo]
                for u in range(2):
                    r = u * g + i
                    t1 = rows_v[r, lo]
                    t2 = rows_v[r, hi]
                    rows_v[r, lo] = t1 * cosv - t2 * sinv
                    rows_v[r, hi] = t1 * sinv + t2 * cosv

    def start_write(t):
        _, rows_v, g, pos0, _, wsem = cfg[t & 1]
        k0, k1 = local_batches(t)
        s0 = pl.ds(pos0, g)
        return [
            pltpu.async_copy(rows_v.at[pl.ds(0, g)],
                             out_ref.at[base + k0, s0], wsem),
            pltpu.async_copy(rows_v.at[pl.ds(g, g)],
                             out_ref.at[base + k1, s0], wsem),
        ]

    gcur = start_gather(0)
    wpend = [None, None]
    for t in range(_TPW):
        p = t & 1
        gnext = None
        if t + 1 < _TPW:
            if wpend[1 - p] is not None:
                for w in wpend[1 - p]:
                    w.wait()
                wpend[1 - p] = None
            gnext = start_gather(t + 1)
        for cp in gcur:
            cp.wait()
        gcur = gnext
        compute(t)
        wpend[p] = start_write(t)
    for ws in wpend:
        if ws is not None:
            for w in ws:
                w.wait()


@jax.jit
def _run(ids0, ids1, table, sin, cos):
    mesh = plsc.VectorSubcoreMesh(core_axis_name="c", subcore_axis_name="s",
                                  num_cores=_NC, num_subcores=_NS)
    f = pl.kernel(
        _body,
        out_type=jax.ShapeDtypeStruct((_B, _S, _DIM), jnp.float32),
        mesh=mesh,
        scratch_types=[
            pltpu.VMEM((_BPW, _G0), jnp.int32),
            pltpu.VMEM((_BPW, _G1), jnp.int32),
            pltpu.VMEM((2 * _G0, _DIM), jnp.float32),
            pltpu.VMEM((2 * _G1, _DIM), jnp.float32),
            pltpu.VMEM((_S, _HALF), jnp.float32),
            pltpu.VMEM((_S, _HALF), jnp.float32),
            pltpu.SemaphoreType.DMA,
            pltpu.SemaphoreType.DMA,
            pltpu.SemaphoreType.DMA,
            pltpu.SemaphoreType.DMA,
        ],
    )
    return f(ids0, ids1, table, sin, cos)


def kernel(ids, table):
    sin, cos = _sincos()
    ids0 = ids[:, :_G0]
    ids1 = ids[:, _G0:]
    return _run(ids0, ids1, table, sin, cos)


# speedup vs baseline: 1.0704x; 1.0704x over previous
import jax
import jax.numpy as jnp
from jax import lax
from jax.experimental import pallas as pl
from jax.experimental.pallas import tpu as pltpu
from jax.experimental.pallas import tpu_sc as plsc

_B = 1024
_S = 200
_DIM = 128
_HALF = _DIM // 2
_BASE = 10000

_NC = 2
_NS = 16
_NW = _NC * _NS
_BPW = _B // _NW
_G0 = 96
_G1 = _S - _G0
_TPW = _BPW


def _sincos():
    inv_freq = 1.0 / (_BASE ** (jnp.arange(0, _HALF, dtype=jnp.float32) / _HALF))
    angles = jnp.arange(_S, dtype=jnp.float32)[:, None] * inv_freq[None, :]
    return jnp.sin(angles), jnp.cos(angles)


def _body(ids0_ref, ids1_ref, table_ref, sin_ref, cos_ref, out_ref,
          idx0_v, idx1_v, rows0_v, rows1_v, sin_v, cos_v,
          gsem0, gsem1, wsem0, wsem1):
    wid = lax.axis_index("s") * _NC + lax.axis_index("c")
    base = wid * _BPW

    pltpu.sync_copy(sin_ref, sin_v)
    pltpu.sync_copy(cos_ref, cos_v)
    pltpu.sync_copy(ids0_ref.at[pl.ds(base, _BPW)], idx0_v)
    pltpu.sync_copy(ids1_ref.at[pl.ds(base, _BPW)], idx1_v)

    cfg = (
        (idx0_v, rows0_v, _G0, 0, gsem0, wsem0),
        (idx1_v, rows1_v, _G1, _G0, gsem1, wsem1),
    )

    def local_batches(t):
        q = t >> 1
        return 2 * q, 2 * q + 1

    def start_gather(t):
        idx_v, rows_v, g, _, gsem, _ = cfg[t & 1]
        k0, k1 = local_batches(t)
        return [
            pltpu.async_copy(table_ref.at[idx_v.at[k]],
                             rows_v.at[pl.ds(c * g, g)], gsem)
            for c, k in enumerate((k0, k1))
        ]

    def compute(t):
        _, rows_v, g, pos0, _, _ = cfg[t & 1]

        @plsc.parallel_loop(0, g, step=1, unroll=2)
        def row_body(i):
            for j in range(_HALF // 16):
                lo = pl.ds(j * 16, 16)
                hi = pl.ds(_HALF + j * 16, 16)
                cosv = cos_v[pos0 + i, lo]
                sinv = sin_v[pos0 + i, lo]
                for u in range(2):
                    r = u * g + i
                    t1 = rows_v[r, lo]
                    t2 = rows_v[r, hi]
                    rows_v[r, lo] = t1 * cosv - t2 * sinv
                    rows_v[r, hi] = t1 * sinv + t2 * cosv

    def start_write(t):
        _, rows_v, g, pos0, _, wsem = cfg[t & 1]
        k0, k1 = local_batches(t)
        s0 = pl.ds(pos0, g)
        return [
            pltpu.async_copy(rows_v.at[pl.ds(0, g)],
                             out_ref.at[base + k0, s0], wsem),
            pltpu.async_copy(rows_v.at[pl.ds(g, g)],
                             out_ref.at[base + k1, s0], wsem),
        ]

    gcur = start_gather(0)
    wpend = [None, None]
    for t in range(_TPW):
        p = t & 1
        gnext = None
        if t + 1 < _TPW:
            if wpend[1 - p] is not None:
                for w in wpend[1 - p]:
                    w.wait()
                wpend[1 - p] = None
            gnext = start_gather(t + 1)
        for cp in gcur:
            cp.wait()
        gcur = gnext
        compute(t)
        wpend[p] = start_write(t)
    for ws in wpend:
        if ws is not None:
            for w in ws:
                w.wait()


@jax.jit
def _run(ids0, ids1, table, sin, cos):
    mesh = plsc.VectorSubcoreMesh(core_axis_name="c", subcore_axis_name="s",
                                  num_cores=_NC, num_subcores=_NS)
    f = pl.kernel(
        _body,
        out_type=jax.ShapeDtypeStruct((_B, _S, _DIM), jnp.float32),
        mesh=mesh,
        scratch_types=[
            pltpu.VMEM((_BPW, _G0), jnp.int32),
            pltpu.VMEM((_BPW, _G1), jnp.int32),
            pltpu.VMEM((2 * _G0, _DIM), jnp.float32),
            pltpu.VMEM((2 * _G1, _DIM), jnp.float32),
            pltpu.VMEM((_S, _HALF), jnp.float32),
            pltpu.VMEM((_S, _HALF), jnp.float32),
            pltpu.SemaphoreType.DMA,
            pltpu.SemaphoreType.DMA,
            pltpu.SemaphoreType.DMA,
            pltpu.SemaphoreType.DMA,
        ],
    )
    return f(ids0, ids1, table, sin, cos)


def kernel(ids, table):
    sin, cos = _sincos()
    ids0 = ids[:, :_G0]
    ids1 = ids[:, _G0:]
    return _run(ids0, ids1, table, sin, cos)
